# baseline (speedup 1.0000x reference)
import functools

import jax
import jax.numpy as jnp
from jax import lax
from jax.experimental import pallas as pl
from jax.experimental.pallas import tpu as pltpu
from jax.experimental.pallas import tpu_sc as plsc

B, S, D = 4096, 50, 64
S1 = S + 1
IDXW = 56

_info = plsc.get_sparse_core_info()
NC, NS = _info.num_cores, _info.num_subcores
NW = NC * NS
RPW = B // NW
NB = 16
NCH = RPW // NB


@functools.partial(
    pl.kernel,
    out_type=jax.ShapeDtypeStruct((B * S1, D), jnp.float32),
    mesh=plsc.VectorSubcoreMesh(core_axis_name="c", subcore_axis_name="s"),
    scratch_types=[
        pltpu.VMEM((NB, IDXW), jnp.int32),
        pltpu.VMEM((NB, D), jnp.float32),
        pltpu.VMEM((NB * S1 + 8, D), jnp.float32),
        pltpu.SemaphoreType.DMA,
    ],
    compiler_params=pltpu.CompilerParams(use_tc_tiling_on_sc=False),
)
def _gather_concat(img_hbm, cap_hbm, table_hbm, out_hbm, idx_v, img_v, out_v, sem):
    wid = lax.axis_index("s") * NC + lax.axis_index("c")

    def chunk(c, carry):
        base = wid * RPW + c * NB
        pltpu.sync_copy(cap_hbm.at[pl.ds(base, NB)], idx_v)
        cps = [
            pltpu.async_copy(
                table_hbm.at[idx_v.at[i]],
                out_v.at[pl.ds(i * S1 + 1, IDXW)],
                sem,
            )
            for i in range(NB)
        ]
        pltpu.sync_copy(img_hbm.at[pl.ds(base, NB)], img_v)
        for cp in cps:
            cp.wait()
        for i in range(NB):
            for k in range(D // 16):
                out_v[i * S1, pl.ds(k * 16, 16)] = img_v[i, pl.ds(k * 16, 16)]
        pltpu.sync_copy(
            out_v.at[pl.ds(0, NB * S1)],
            out_hbm.at[pl.ds(base * S1, NB * S1)],
        )
        return carry

    lax.fori_loop(0, NCH, chunk, 0)


def kernel(image_features, captions, embedding_table):
    cap = captions.astype(jnp.int32)
    nxt = jnp.concatenate([cap[1:, :5], jnp.zeros((1, 5), jnp.int32)], axis=0)
    capp = jnp.concatenate([cap, jnp.zeros((B, 1), jnp.int32), nxt], axis=1)
    out = _gather_concat(image_features, capp, embedding_table)
    return out.reshape(B, S1, D)
